# Initial kernel scaffold; baseline (speedup 1.0000x reference)
import jax
import jax.numpy as jnp
from jax.experimental import pallas as pl


def kernel(x1, x2, table):
    raise NotImplementedError("write your pallas kernel here")


# speedup vs baseline: 6.7448x; 6.7448x over previous
import jax
import jax.numpy as jnp
from jax.experimental import pallas as pl


_BLK = 2048


def _body(x1_ref, x2_ref, table_ref, out_ref):
    idx = x1_ref[0, 0, :]
    iota = jax.lax.broadcasted_iota(jnp.int32, (64, _BLK), 0)
    onehot = (iota == idx[None, :]).astype(jnp.float32)
    emb = jax.lax.dot_general(
        onehot, table_ref[...],
        dimension_numbers=(((0,), (0,)), ((), ())),
        preferred_element_type=jnp.float32,
    )
    out_ref[...] = emb * x2_ref[...]


def kernel(x1, x2, table):
    B, L = x1.shape
    D = x2.shape[-1]
    N = B * L
    nblk = N // _BLK
    x1f = x1.reshape(nblk, 1, _BLK).astype(jnp.int32)
    x2f = x2.reshape(N, D)

    out = pl.pallas_call(
        _body,
        grid=(nblk,),
        in_specs=[
            pl.BlockSpec((1, 1, _BLK), lambda i: (i, 0, 0)),
            pl.BlockSpec((_BLK, D), lambda i: (i, 0)),
            pl.BlockSpec((64, D), lambda i: (0, 0)),
        ],
        out_specs=pl.BlockSpec((_BLK, D), lambda i: (i, 0)),
        out_shape=jax.ShapeDtypeStruct((N, D), jnp.float32),
    )(x1f, x2f, table)
    return out.reshape(B, L, D)


# speedup vs baseline: 10.9648x; 1.6257x over previous
import jax
import jax.numpy as jnp
from jax.experimental import pallas as pl


_BLK = 8192


def _body(x1_ref, x2_ref, table_ref, out_ref):
    idx = x1_ref[0, 0, :]
    iota = jax.lax.broadcasted_iota(jnp.int32, (64, _BLK), 0)
    onehot = (iota == idx[None, :]).astype(jnp.float32)
    emb = jax.lax.dot_general(
        onehot, table_ref[...],
        dimension_numbers=(((0,), (0,)), ((), ())),
        preferred_element_type=jnp.float32,
    )
    out_ref[...] = emb * x2_ref[...]


def kernel(x1, x2, table):
    B, L = x1.shape
    D = x2.shape[-1]
    N = B * L
    nblk = N // _BLK
    x1f = x1.reshape(nblk, 1, _BLK).astype(jnp.int32)
    x2f = x2.reshape(N, D)

    out = pl.pallas_call(
        _body,
        grid=(nblk,),
        in_specs=[
            pl.BlockSpec((1, 1, _BLK), lambda i: (i, 0, 0)),
            pl.BlockSpec((_BLK, D), lambda i: (i, 0)),
            pl.BlockSpec((64, D), lambda i: (0, 0)),
        ],
        out_specs=pl.BlockSpec((_BLK, D), lambda i: (i, 0)),
        out_shape=jax.ShapeDtypeStruct((N, D), jnp.float32),
    )(x1f, x2f, table)
    return out.reshape(B, L, D)


# speedup vs baseline: 11.2109x; 1.0224x over previous
import jax
import jax.numpy as jnp
from jax.experimental import pallas as pl


_BLK = 16384


def _body(x1_ref, x2_ref, table_ref, out_ref):
    idx = x1_ref[0, 0, :]
    iota = jax.lax.broadcasted_iota(jnp.int32, (64, _BLK), 0)
    onehot = (iota == idx[None, :]).astype(jnp.float32)
    emb = jax.lax.dot_general(
        onehot, table_ref[...],
        dimension_numbers=(((0,), (0,)), ((), ())),
        preferred_element_type=jnp.float32,
    )
    out_ref[...] = emb * x2_ref[...]


def kernel(x1, x2, table):
    B, L = x1.shape
    D = x2.shape[-1]
    N = B * L
    nblk = N // _BLK
    x1f = x1.reshape(nblk, 1, _BLK).astype(jnp.int32)
    x2f = x2.reshape(N, D)

    out = pl.pallas_call(
        _body,
        grid=(nblk,),
        in_specs=[
            pl.BlockSpec((1, 1, _BLK), lambda i: (i, 0, 0)),
            pl.BlockSpec((_BLK, D), lambda i: (i, 0)),
            pl.BlockSpec((64, D), lambda i: (0, 0)),
        ],
        out_specs=pl.BlockSpec((_BLK, D), lambda i: (i, 0)),
        out_shape=jax.ShapeDtypeStruct((N, D), jnp.float32),
    )(x1f, x2f, table)
    return out.reshape(B, L, D)
